# baseline (speedup 1.0000x reference)
import jax
import jax.numpy as jnp
from jax import lax
from jax.experimental import pallas as pl
from jax.experimental.pallas import tpu as pltpu
import functools

_NEG = -1.0e30
_GROUP = 16
_SEQ_BLOCK = 128
_N_TILE = 2048


def _ln(x, g, b, eps=1e-5):
    mu = jnp.mean(x, axis=-1, keepdims=True)
    var = jnp.mean(jnp.square(x - mu), axis=-1, keepdims=True)
    return (x - mu) * lax.rsqrt(var + eps) * g + b


def _group_chain(emb_b, mc, lens, amat, wv, w1, w2, rows, *, L):
    f32 = jnp.float32
    bf16 = jnp.bfloat16
    (ln1g, ln1b, a_row, u_row, c0b, bv, ln2g, ln2b,
     b1, b2, ffg, ffb, ln3g, ln3b) = rows
    M, D = emb_b.shape
    G = M // L

    row0 = lax.broadcasted_iota(jnp.int32, (G, 1), 0) * L
    tgt = row0 + lens - 1
    cols = lax.broadcasted_iota(jnp.int32, (G, M), 1)
    ohm = jnp.where(cols == tgt, 1.0, 0.0) * mc

    seq_g = jnp.dot(ohm.astype(bf16), emb_b,
                    preferred_element_type=f32)
    mask_g = jnp.sum(ohm, axis=-1, keepdims=True)

    q_in = _ln(seq_g, ln1g, ln1b)
    t = jnp.dot(q_in.astype(bf16), amat,
                preferred_element_type=f32) + a_row
    qb = jnp.sum(q_in * u_row + c0b, axis=-1, keepdims=True)

    tcat = jnp.concatenate([t.astype(bf16),
                            jnp.ones((8, D), bf16)], axis=0)
    sc_cs = lax.dot_general(tcat, emb_b, (((1,), (1,)), ((), ())),
                            preferred_element_type=f32)
    scores = sc_cs[0:G] + qb
    colsum = sc_cs[G:G + 1] * mc

    allowed = jnp.logical_and(cols >= row0, cols <= tgt)
    live = jnp.logical_and(allowed, colsum != 0.0)
    sc = jnp.where(live, scores, _NEG)
    m = jnp.max(sc, axis=-1, keepdims=True)
    p = jnp.exp(sc - m)
    denom = jnp.sum(p, axis=-1, keepdims=True)
    pseq = jnp.dot((p * mc).astype(bf16), emb_b,
                   preferred_element_type=f32)
    num = jnp.dot(pseq.astype(bf16), wv,
                  preferred_element_type=f32) + denom * bv
    qmask = jnp.sign(jnp.abs(jnp.sum(q_in, axis=-1, keepdims=True)))
    mh = num * (1.0 / denom) * qmask + q_in

    x2 = _ln(mh, ln2g, ln2b)
    h1 = jnp.maximum(jnp.dot(x2.astype(bf16), w1,
                             preferred_element_type=f32) + b1, 0.0)
    h2 = jnp.dot(h1.astype(bf16), w2,
                 preferred_element_type=f32) + b2
    ffi = _ln(h2 + x2, ffg, ffb)
    return _ln(ffi * mask_g, ln3g, ln3b)


def _state_kernel(emb_ref, maskc_ref, len_ref, w_ref, p_ref, h_ref, *, L, S):
    bf16 = jnp.bfloat16
    M, D = emb_ref.shape

    w = w_ref[...]
    amat, wv, w1, w2 = (w[0:D], w[D:2 * D], w[2 * D:3 * D], w[3 * D:4 * D])
    p_ = p_ref[...]
    rows = tuple(p_[i:i + 1] for i in range(14))

    Mg = _GROUP * L
    for h in range(S // _GROUP):
        emb_b = emb_ref[h * Mg:(h + 1) * Mg, :].astype(bf16)
        mc = maskc_ref[:, h * Mg:(h + 1) * Mg]
        lens = len_ref[h * _GROUP:(h + 1) * _GROUP, :]
        h_ref[h * _GROUP:(h + 1) * _GROUP, :] = _group_chain(
            emb_b, mc, lens, amat, wv, w1, w2, rows, L=L)


def _logits_kernel(h_ref, w_ref, b_ref, out_ref):
    hb = h_ref[...].astype(jnp.bfloat16)
    wb = w_ref[...].astype(jnp.bfloat16)
    out_ref[...] = (jnp.dot(hb, wb, preferred_element_type=jnp.float32)
                    + b_ref[...])


def kernel(inputs_emb, mask, len_states,
           ln1_g, ln1_b, wq, bq, wk, bk, wv, bv,
           ln2_g, ln2_b, w1, b1, w2, b2,
           ffln_g, ffln_b, ln3_g, ln3_b, sfc_w, sfc_b):
    B, L, D = inputs_emb.shape
    N = sfc_w.shape[1]
    bf16 = jnp.bfloat16

    S = _SEQ_BLOCK
    B_pad = ((B + S - 1) // S) * S
    len_states = len_states.astype(jnp.int32)
    if B_pad != B:
        pad = B_pad - B
        inputs_emb = jnp.pad(inputs_emb, ((0, pad), (0, 0), (0, 0)))
        mask = jnp.pad(mask, ((0, pad), (0, 0), (0, 0)))
        len_states = jnp.concatenate([len_states, jnp.ones((pad,), jnp.int32)])

    emb_flat = inputs_emb.reshape(B_pad * L, D)
    maskcol = mask.reshape(1, B_pad * L)
    len2d = len_states.reshape(B_pad, 1)

    scale = 1.0 / (float(D) ** 0.5)
    amat = (wq @ wk.T) * scale
    a_row = (bq @ wk.T) * scale
    u_row = (bk @ wq.T) * scale
    c0b = jnp.full((1, D), scale * jnp.sum(bq * bk) / D, jnp.float32)
    wpack = jnp.concatenate([amat, wv, w1, w2], axis=0).astype(bf16)
    ppack = jnp.concatenate(
        [ln1_g, ln1_b, a_row, u_row, c0b, bv, ln2_g, ln2_b,
         b1, b2, ffln_g, ffln_b, ln3_g, ln3_b,
         jnp.zeros((2, D), jnp.float32)], axis=0)

    state = pl.pallas_call(
        functools.partial(_state_kernel, L=L, S=S),
        out_shape=jax.ShapeDtypeStruct((B_pad, D), jnp.float32),
        grid=(B_pad // S,),
        in_specs=[pl.BlockSpec((S * L, D), lambda g: (g, 0)),
                  pl.BlockSpec((1, S * L), lambda g: (0, g)),
                  pl.BlockSpec((S, 1), lambda g: (g, 0)),
                  pl.BlockSpec(wpack.shape, lambda g: (0, 0)),
                  pl.BlockSpec(ppack.shape, lambda g: (0, 0))],
        out_specs=pl.BlockSpec((S, D), lambda g: (g, 0)),
        compiler_params=pltpu.CompilerParams(
            dimension_semantics=("arbitrary",)),
    )(emb_flat, maskcol, len2d, wpack, ppack)

    nt = _N_TILE
    while N % nt:
        nt //= 2
    nt = max(nt, 128)
    N_pad = ((N + nt - 1) // nt) * nt
    if N_pad != N:
        sfc_w = jnp.pad(sfc_w, ((0, 0), (0, N_pad - N)))
        sfc_b = jnp.pad(sfc_b, ((0, 0), (0, N_pad - N)))

    logits = pl.pallas_call(
        _logits_kernel,
        out_shape=jax.ShapeDtypeStruct((B_pad, N_pad), jnp.float32),
        grid=(N_pad // nt,),
        in_specs=[pl.BlockSpec((B_pad, D), lambda n: (0, 0)),
                  pl.BlockSpec((D, nt), lambda n: (0, n)),
                  pl.BlockSpec((1, nt), lambda n: (0, n))],
        out_specs=pl.BlockSpec((B_pad, nt), lambda n: (0, n)),
        compiler_params=pltpu.CompilerParams(
            dimension_semantics=("arbitrary",)),
    )(state, sfc_w, sfc_b)

    return logits[:B, :N]


# speedup vs baseline: 1.6279x; 1.6279x over previous
import jax
import jax.numpy as jnp
from jax import lax
from jax.experimental import pallas as pl
from jax.experimental.pallas import tpu as pltpu
import functools

_NEG = -1.0e30
_GROUP = 16
_SEQ_BLOCK = 128
_N_TILE = 2048


def _ln(x, g, b, eps=1e-5):
    mu = jnp.mean(x, axis=-1, keepdims=True)
    var = jnp.mean(jnp.square(x - mu), axis=-1, keepdims=True)
    return (x - mu) * lax.rsqrt(var + eps) * g + b


def _state_kernel(emb_ref, maskc_ref, len_ref, w_ref, p_ref,
                  h_ref, mg_ref, *, L, S):
    f32 = jnp.float32
    bf16 = jnp.bfloat16
    M, D = emb_ref.shape

    emb_b = emb_ref[...].astype(bf16)
    mc = maskc_ref[...]
    w = w_ref[...]
    amat, wv, w1, w2 = (w[0:D], w[D:2 * D], w[2 * D:3 * D], w[3 * D:4 * D])
    p_ = p_ref[...]
    (ln1g, ln1b, a_row, u_row, c0b, bv, ln2g, ln2b,
     b1, b2, ffg, ffb, ln3g, ln3b) = [p_[i:i + 1] for i in range(14)]

    lens = len_ref[...]
    row0 = lax.broadcasted_iota(jnp.int32, (S, 1), 0) * L
    tgt = row0 + lens - 1
    cols = lax.broadcasted_iota(jnp.int32, (S, M), 1)
    ohm = jnp.where(cols == tgt, 1.0, 0.0) * mc

    seq_g = jnp.dot(ohm.astype(bf16), emb_b,
                    preferred_element_type=f32)
    mask_g = jnp.sum(ohm, axis=-1, keepdims=True)

    q_in = _ln(seq_g, ln1g, ln1b)
    t = jnp.dot(q_in.astype(bf16), amat,
                preferred_element_type=f32) + a_row
    qb = jnp.sum(q_in * u_row + c0b, axis=-1, keepdims=True)

    tcat = jnp.concatenate([t.astype(bf16),
                            jnp.ones((8, D), bf16)], axis=0)
    sc_cs = lax.dot_general(tcat, emb_b, (((1,), (1,)), ((), ())),
                            preferred_element_type=f32)
    scores = sc_cs[0:S] + qb
    colsum = sc_cs[S:S + 1] * mc

    allowed = jnp.logical_and(cols >= row0, cols <= tgt)
    live = jnp.logical_and(allowed, colsum != 0.0)
    sc = jnp.where(live, scores, _NEG)
    m = jnp.max(sc, axis=-1, keepdims=True)
    p = jnp.exp(sc - m)
    if S > _GROUP:
        samegroup = (cols // (_GROUP * L)) == (
            lax.broadcasted_iota(jnp.int32, (S, 1), 0) // _GROUP)
        p = jnp.where(jnp.logical_or(m > 0.5 * _NEG, samegroup), p, 0.0)
    denom = jnp.sum(p, axis=-1, keepdims=True)
    pseq = jnp.dot((p * mc).astype(bf16), emb_b,
                   preferred_element_type=f32)
    num = jnp.dot(pseq.astype(bf16), wv,
                  preferred_element_type=f32) + denom * bv
    qmask = jnp.sign(jnp.abs(jnp.sum(q_in, axis=-1, keepdims=True)))
    h_ref[...] = num * (1.0 / denom) * qmask + q_in
    mg_ref[...] = mask_g


def _logits_kernel(mh_ref, mg_ref, w_ref, p_ref, sw_ref, sb_ref,
                   out_ref, state_ref):
    f32 = jnp.float32
    bf16 = jnp.bfloat16
    D = mh_ref.shape[1]

    @pl.when(pl.program_id(0) == 0)
    def _tail():
        w = w_ref[...]
        w1, w2 = w[2 * D:3 * D], w[3 * D:4 * D]
        p_ = p_ref[...]
        (ln2g, ln2b, b1, b2, ffg, ffb, ln3g, ln3b) = [
            p_[i:i + 1] for i in range(6, 14)]
        x2 = _ln(mh_ref[...], ln2g, ln2b)
        h1 = jnp.maximum(jnp.dot(x2.astype(bf16), w1,
                                 preferred_element_type=f32) + b1, 0.0)
        h2 = jnp.dot(h1.astype(bf16), w2,
                     preferred_element_type=f32) + b2
        ffi = _ln(h2 + x2, ffg, ffb)
        state_ref[...] = _ln(ffi * mg_ref[...], ln3g, ln3b).astype(bf16)

    wb = sw_ref[...].astype(bf16)
    out_ref[...] = (jnp.dot(state_ref[...], wb,
                            preferred_element_type=f32) + sb_ref[...])


def kernel(inputs_emb, mask, len_states,
           ln1_g, ln1_b, wq, bq, wk, bk, wv, bv,
           ln2_g, ln2_b, w1, b1, w2, b2,
           ffln_g, ffln_b, ln3_g, ln3_b, sfc_w, sfc_b):
    B, L, D = inputs_emb.shape
    N = sfc_w.shape[1]
    bf16 = jnp.bfloat16

    S = _SEQ_BLOCK
    B_pad = ((B + S - 1) // S) * S
    len_states = len_states.astype(jnp.int32)
    if B_pad != B:
        pad = B_pad - B
        inputs_emb = jnp.pad(inputs_emb, ((0, pad), (0, 0), (0, 0)))
        mask = jnp.pad(mask, ((0, pad), (0, 0), (0, 0)))
        len_states = jnp.concatenate([len_states, jnp.ones((pad,), jnp.int32)])

    emb_flat = inputs_emb.reshape(B_pad * L, D)
    maskcol = mask.reshape(1, B_pad * L)
    len2d = len_states.reshape(B_pad, 1)

    scale = 1.0 / (float(D) ** 0.5)
    amat = (wq @ wk.T) * scale
    a_row = (bq @ wk.T) * scale
    u_row = (bk @ wq.T) * scale
    c0b = jnp.full((1, D), scale * jnp.sum(bq * bk) / D, jnp.float32)
    wpack = jnp.concatenate([amat, wv, w1, w2], axis=0).astype(bf16)
    ppack = jnp.concatenate(
        [ln1_g, ln1_b, a_row, u_row, c0b, bv, ln2_g, ln2_b,
         b1, b2, ffln_g, ffln_b, ln3_g, ln3_b,
         jnp.zeros((2, D), jnp.float32)], axis=0)

    mh, mg = pl.pallas_call(
        functools.partial(_state_kernel, L=L, S=S),
        out_shape=(jax.ShapeDtypeStruct((B_pad, D), jnp.float32),
                   jax.ShapeDtypeStruct((B_pad, 1), jnp.float32)),
        grid=(B_pad // S,),
        in_specs=[pl.BlockSpec((S * L, D), lambda g: (g, 0)),
                  pl.BlockSpec((1, S * L), lambda g: (0, g)),
                  pl.BlockSpec((S, 1), lambda g: (g, 0)),
                  pl.BlockSpec(wpack.shape, lambda g: (0, 0)),
                  pl.BlockSpec(ppack.shape, lambda g: (0, 0))],
        out_specs=(pl.BlockSpec((S, D), lambda g: (g, 0)),
                   pl.BlockSpec((S, 1), lambda g: (g, 0))),
        compiler_params=pltpu.CompilerParams(
            dimension_semantics=("arbitrary",)),
    )(emb_flat, maskcol, len2d, wpack, ppack)

    nt = _N_TILE
    while N % nt:
        nt //= 2
    nt = max(nt, 128)
    N_pad = ((N + nt - 1) // nt) * nt
    if N_pad != N:
        sfc_w = jnp.pad(sfc_w, ((0, 0), (0, N_pad - N)))
        sfc_b = jnp.pad(sfc_b, ((0, 0), (0, N_pad - N)))

    logits = pl.pallas_call(
        _logits_kernel,
        out_shape=jax.ShapeDtypeStruct((B_pad, N_pad), jnp.float32),
        grid=(N_pad // nt,),
        in_specs=[pl.BlockSpec((B_pad, D), lambda n: (0, 0)),
                  pl.BlockSpec((B_pad, 1), lambda n: (0, 0)),
                  pl.BlockSpec(wpack.shape, lambda n: (0, 0)),
                  pl.BlockSpec(ppack.shape, lambda n: (0, 0)),
                  pl.BlockSpec((D, nt), lambda n: (0, n)),
                  pl.BlockSpec((1, nt), lambda n: (0, n))],
        out_specs=pl.BlockSpec((B_pad, nt), lambda n: (0, n)),
        scratch_shapes=[pltpu.VMEM((B_pad, D), jnp.bfloat16)],
        compiler_params=pltpu.CompilerParams(
            dimension_semantics=("arbitrary",)),
    )(mh, mg, wpack, ppack, sfc_w, sfc_b)

    return logits[:B, :N]


# speedup vs baseline: 1.6926x; 1.0397x over previous
import jax
import jax.numpy as jnp
from jax import lax
from jax.experimental import pallas as pl
from jax.experimental.pallas import tpu as pltpu
import functools

_NEG = -1.0e30
_GROUP = 16
_SEQ_BLOCK = 128
_N_TILE = 2048


def _ln(x, g, b, eps=1e-5):
    mu = jnp.mean(x, axis=-1, keepdims=True)
    var = jnp.mean(jnp.square(x - mu), axis=-1, keepdims=True)
    return (x - mu) * lax.rsqrt(var + eps) * g + b


def _state_kernel(emb_ref, maskc_ref, len_ref, w_ref, p_ref,
                  h_ref, mg_ref, *, L, S, scale):
    f32 = jnp.float32
    bf16 = jnp.bfloat16
    M, D = emb_ref.shape

    emb_b = emb_ref[...].astype(bf16)
    mc = maskc_ref[...]
    w = w_ref[...]
    wq, wk, wv = w[0:D], w[D:2 * D], w[2 * D:3 * D]
    p_ = p_ref[...]
    (ln1g, ln1b, bq, bk, bv) = [p_[i:i + 1] for i in range(5)]

    amat = (lax.dot_general(wq, wk, (((1,), (1,)), ((), ())),
                            preferred_element_type=f32) * scale).astype(bf16)
    a_row = lax.dot_general(bq.astype(bf16), wk, (((1,), (1,)), ((), ())),
                            preferred_element_type=f32) * scale
    u_row = lax.dot_general(bk.astype(bf16), wq, (((1,), (1,)), ((), ())),
                            preferred_element_type=f32) * scale
    c0b = bq * bk * scale

    lens = len_ref[...]
    row0 = lax.broadcasted_iota(jnp.int32, (S, 1), 0) * L
    tgt = row0 + lens - 1
    cols = lax.broadcasted_iota(jnp.int32, (S, M), 1)
    ohm = jnp.where(cols == tgt, 1.0, 0.0) * mc

    seq_g = jnp.dot(ohm.astype(bf16), emb_b,
                    preferred_element_type=f32)
    mask_g = jnp.sum(ohm, axis=-1, keepdims=True)

    q_in = _ln(seq_g, ln1g, ln1b)
    t = jnp.dot(q_in.astype(bf16), amat,
                preferred_element_type=f32) + a_row
    qb = jnp.sum(q_in * u_row + c0b, axis=-1, keepdims=True)

    tcat = jnp.concatenate([t.astype(bf16),
                            jnp.ones((8, D), bf16)], axis=0)
    sc_cs = lax.dot_general(tcat, emb_b, (((1,), (1,)), ((), ())),
                            preferred_element_type=f32)
    scores = sc_cs[0:S] + qb
    colsum = sc_cs[S:S + 1] * mc

    allowed = jnp.logical_and(cols >= row0, cols <= tgt)
    live = jnp.logical_and(allowed, colsum != 0.0)
    sc = jnp.where(live, scores, _NEG)
    m = jnp.max(sc, axis=-1, keepdims=True)
    p = jnp.exp(sc - m)
    if S > _GROUP:
        samegroup = (cols // (_GROUP * L)) == (
            lax.broadcasted_iota(jnp.int32, (S, 1), 0) // _GROUP)
        p = jnp.where(jnp.logical_or(m > 0.5 * _NEG, samegroup), p, 0.0)
    denom = jnp.sum(p, axis=-1, keepdims=True)
    pseq = jnp.dot((p * mc).astype(bf16), emb_b,
                   preferred_element_type=f32)
    num = jnp.dot(pseq.astype(bf16), wv,
                  preferred_element_type=f32) + denom * bv
    qmask = jnp.sign(jnp.abs(jnp.sum(q_in, axis=-1, keepdims=True)))
    h_ref[...] = num * (1.0 / denom) * qmask + q_in
    mg_ref[...] = mask_g


def _logits_kernel(mh_ref, mg_ref, w_ref, p_ref, sw_ref, sb_ref,
                   out_ref, state_ref):
    f32 = jnp.float32
    bf16 = jnp.bfloat16
    D = mh_ref.shape[1]

    @pl.when(pl.program_id(0) == 0)
    def _tail():
        w = w_ref[...]
        w1, w2 = w[3 * D:4 * D], w[4 * D:5 * D]
        p_ = p_ref[...]
        (ln2g, ln2b, b1, b2, ffg, ffb, ln3g, ln3b) = [
            p_[i:i + 1] for i in range(5, 13)]
        x2 = _ln(mh_ref[...], ln2g, ln2b)
        h1 = jnp.maximum(jnp.dot(x2.astype(bf16), w1,
                                 preferred_element_type=f32) + b1, 0.0)
        h2 = jnp.dot(h1.astype(bf16), w2,
                     preferred_element_type=f32) + b2
        ffi = _ln(h2 + x2, ffg, ffb)
        state_ref[...] = _ln(ffi * mg_ref[...], ln3g, ln3b).astype(bf16)

    wb = sw_ref[...].astype(bf16)
    out_ref[...] = (jnp.dot(state_ref[...], wb,
                            preferred_element_type=f32) + sb_ref[...])


def kernel(inputs_emb, mask, len_states,
           ln1_g, ln1_b, wq, bq, wk, bk, wv, bv,
           ln2_g, ln2_b, w1, b1, w2, b2,
           ffln_g, ffln_b, ln3_g, ln3_b, sfc_w, sfc_b):
    B, L, D = inputs_emb.shape
    N = sfc_w.shape[1]
    bf16 = jnp.bfloat16

    S = _SEQ_BLOCK
    B_pad = ((B + S - 1) // S) * S
    len_states = len_states.astype(jnp.int32)
    if B_pad != B:
        pad = B_pad - B
        inputs_emb = jnp.pad(inputs_emb, ((0, pad), (0, 0), (0, 0)))
        mask = jnp.pad(mask, ((0, pad), (0, 0), (0, 0)))
        len_states = jnp.concatenate([len_states, jnp.ones((pad,), jnp.int32)])

    emb_flat = inputs_emb.reshape(B_pad * L, D)
    maskcol = mask.reshape(1, B_pad * L)
    len2d = len_states.reshape(B_pad, 1)

    scale = 1.0 / (float(D) ** 0.5)
    wpack = jnp.concatenate([wq, wk, wv, w1, w2], axis=0).astype(bf16)
    ppack = jnp.concatenate(
        [ln1_g, ln1_b, bq, bk, bv, ln2_g, ln2_b,
         b1, b2, ffln_g, ffln_b, ln3_g, ln3_b,
         jnp.zeros((3, D), jnp.float32)], axis=0)

    mh, mg = pl.pallas_call(
        functools.partial(_state_kernel, L=L, S=S, scale=scale),
        out_shape=(jax.ShapeDtypeStruct((B_pad, D), jnp.float32),
                   jax.ShapeDtypeStruct((B_pad, 1), jnp.float32)),
        grid=(B_pad // S,),
        in_specs=[pl.BlockSpec((S * L, D), lambda g: (g, 0)),
                  pl.BlockSpec((1, S * L), lambda g: (0, g)),
                  pl.BlockSpec((S, 1), lambda g: (g, 0)),
                  pl.BlockSpec(wpack.shape, lambda g: (0, 0)),
                  pl.BlockSpec(ppack.shape, lambda g: (0, 0))],
        out_specs=(pl.BlockSpec((S, D), lambda g: (g, 0)),
                   pl.BlockSpec((S, 1), lambda g: (g, 0))),
        compiler_params=pltpu.CompilerParams(
            dimension_semantics=("arbitrary",)),
    )(emb_flat, maskcol, len2d, wpack, ppack)

    nt = _N_TILE
    while N % nt:
        nt //= 2
    nt = max(nt, 128)
    N_pad = ((N + nt - 1) // nt) * nt
    if N_pad != N:
        sfc_w = jnp.pad(sfc_w, ((0, 0), (0, N_pad - N)))
        sfc_b = jnp.pad(sfc_b, ((0, 0), (0, N_pad - N)))

    logits = pl.pallas_call(
        _logits_kernel,
        out_shape=jax.ShapeDtypeStruct((B_pad, N_pad), jnp.float32),
        grid=(N_pad // nt,),
        in_specs=[pl.BlockSpec((B_pad, D), lambda n: (0, 0)),
                  pl.BlockSpec((B_pad, 1), lambda n: (0, 0)),
                  pl.BlockSpec(wpack.shape, lambda n: (0, 0)),
                  pl.BlockSpec(ppack.shape, lambda n: (0, 0)),
                  pl.BlockSpec((D, nt), lambda n: (0, n)),
                  pl.BlockSpec((1, nt), lambda n: (0, n))],
        out_specs=pl.BlockSpec((B_pad, nt), lambda n: (0, n)),
        scratch_shapes=[pltpu.VMEM((B_pad, D), jnp.bfloat16)],
        compiler_params=pltpu.CompilerParams(
            dimension_semantics=("arbitrary",)),
    )(mh, mg, wpack, ppack, sfc_w, sfc_b)

    return logits[:B, :N]
